# baseline (speedup 1.0000x reference)
import functools

import jax
import jax.numpy as jnp
from jax import lax
from jax.experimental import pallas as pl
from jax.experimental.pallas import tpu as pltpu
from jax.experimental.pallas import tpu_sc as plsc

EMBED = 32
POS_V = 64
BATCH = 4096
SEQ = 128

NC, NS = 2, 16
NW = NC * NS
SEQ_PER_W = BATCH // NW
CHUNK = 8
NCHUNK = SEQ_PER_W // CHUNK
NLANE = 16


def _add_pos(rows_v, pos_v):
    def body(k, _):
        for h in range(EMBED // NLANE):
            sl = pl.ds(h * NLANE, NLANE)
            p = pos_v[k, sl]
            for j in range(CHUNK):
                rows_v[j, k, sl] += p
        return 0

    lax.fori_loop(0, SEQ, body, 0, unroll=False)


def _sc_body(tok_hbm, pos_hbm, patch_hbm, out_hbm, idx_v, pos_v, rows_v,
             gsems, ssems):
    wid = lax.axis_index("s") * NC + lax.axis_index("c")
    seq0 = wid * SEQ_PER_W

    pltpu.sync_copy(patch_hbm.at[pl.ds(seq0, SEQ_PER_W)], idx_v)
    pltpu.sync_copy(pos_hbm, pos_v.at[pl.ds(0, POS_V)])
    for h in range(EMBED // NLANE):
        sl = pl.ds(h * NLANE, NLANE)
        last = pos_v[POS_V - 1, sl]
        for k in range(POS_V, SEQ):
            pos_v[k, sl] = last

    gathers = [None, None]
    stores = [None, None]
    for c in range(NCHUNK + 1):
        b = c % 2
        if c < NCHUNK:
            if stores[b] is not None:
                stores[b].wait()
                stores[b] = None
            gathers[b] = [
                pltpu.async_copy(
                    tok_hbm.at[idx_v.at[c * CHUNK + j]],
                    rows_v.at[b, j],
                    gsems.at[b],
                )
                for j in range(CHUNK)
            ]
        if c >= 1:
            b2 = (c - 1) % 2
            for cp in gathers[b2]:
                cp.wait()
            _add_pos(rows_v.at[b2], pos_v)
            stores[b2] = pltpu.async_copy(
                rows_v.at[b2],
                out_hbm.at[pl.ds(seq0 + (c - 1) * CHUNK, CHUNK)],
                ssems.at[b2],
            )
    for st in stores:
        if st is not None:
            st.wait()


@functools.partial(
    pl.kernel,
    out_type=jax.ShapeDtypeStruct((BATCH, SEQ, EMBED), jnp.float32),
    mesh=plsc.VectorSubcoreMesh(core_axis_name="c", subcore_axis_name="s"),
    scratch_types=[
        pltpu.VMEM((SEQ_PER_W, SEQ), jnp.int32),
        pltpu.VMEM((SEQ, EMBED), jnp.float32),
        pltpu.VMEM((2, CHUNK, SEQ, EMBED), jnp.float32),
        pltpu.SemaphoreType.DMA((2,)),
        pltpu.SemaphoreType.DMA((2,)),
    ],
    compiler_params=pltpu.CompilerParams(use_tc_tiling_on_sc=False),
)
def _sc_embed(tok_hbm, pos_hbm, patch_hbm, out_hbm, idx_v, pos_v, rows_v,
              gsems, ssems):
    _sc_body(tok_hbm, pos_hbm, patch_hbm, out_hbm, idx_v, pos_v, rows_v,
             gsems, ssems)


def kernel(patches, token_table, pos_table):
    patches = patches.astype(jnp.int32)
    return _sc_embed(token_table, pos_table, patches)
